# Initial kernel scaffold; baseline (speedup 1.0000x reference)
import jax
import jax.numpy as jnp
from jax.experimental import pallas as pl


def kernel(x, wg, W_gu_shared, W_down_shared, W_gu_exp, W_down_exp):
    raise NotImplementedError("write your pallas kernel here")


# speedup vs baseline: 1.7033x; 1.7033x over previous
import jax
import jax.numpy as jnp
from jax.experimental import pallas as pl

_T, _D, _E, _K, _I = 2048, 768, 64, 8, 6144
_H = _I // 2
_FB = 512
_NF = _H // _FB


def _gating_kernel(x_ref, wg_ref, comb_ref):
    x = x_ref[...]
    wg = wg_ref[...]
    logits = jax.lax.dot_general(
        x, wg, (((1,), (1,)), ((), ())), preferred_element_type=jnp.float32)
    m = jnp.max(logits, axis=1, keepdims=True)
    p = jnp.exp(logits - m)
    gates = p / jnp.sum(p, axis=1, keepdims=True)

    g = gates
    comb = jnp.zeros_like(gates)
    topsum = jnp.zeros((_T, 1), jnp.float32)
    iota = jax.lax.broadcasted_iota(jnp.int32, (_T, _E), 1)
    for _ in range(_K):
        mx = jnp.max(g, axis=1, keepdims=True)
        sel = g == mx
        first = jnp.min(jnp.where(sel, iota, _E), axis=1, keepdims=True)
        onehot = iota == first
        comb = comb + jnp.where(onehot, gates, 0.0)
        topsum = topsum + mx
        g = jnp.where(onehot, -jnp.inf, g)
    denom = jnp.maximum(topsum, jnp.finfo(jnp.float32).eps)
    comb_ref[...] = comb / denom


def _moe_kernel(comb_ref, x_ref, wg_ref, wu_ref, wd_ref, out_ref):
    e = pl.program_id(0)
    f = pl.program_id(1)

    @pl.when(jnp.logical_and(e == 0, f == 0))
    def _():
        out_ref[...] = jnp.zeros_like(out_ref)

    x = x_ref[...]
    x1 = jax.lax.dot(x, wg_ref[0], preferred_element_type=jnp.float32)
    x2 = jax.lax.dot(x, wu_ref[0], preferred_element_type=jnp.float32)
    act = x1 * (x2 * jax.nn.sigmoid(x2))

    iota = jax.lax.broadcasted_iota(jnp.int32, (_T, _E), 1)
    w = jnp.sum(jnp.where(iota == e, comb_ref[...], 0.0), axis=1,
                keepdims=True)
    act = act * w
    out_ref[...] += jax.lax.dot(act, wd_ref[0],
                                preferred_element_type=jnp.float32)


def _shared_kernel(x_ref, wg_ref, wu_ref, wd_ref, out_ref):
    f = pl.program_id(0)

    @pl.when(f == 0)
    def _():
        out_ref[...] = jnp.zeros_like(out_ref)

    x = x_ref[...]
    x1 = jax.lax.dot(x, wg_ref[...], preferred_element_type=jnp.float32)
    x2 = jax.lax.dot(x, wu_ref[...], preferred_element_type=jnp.float32)
    act = x1 * (x2 * jax.nn.sigmoid(x2))
    out_ref[...] += jax.lax.dot(act, wd_ref[...],
                                preferred_element_type=jnp.float32)


def kernel(x, wg, W_gu_shared, W_down_shared, W_gu_exp, W_down_exp):
    comb = pl.pallas_call(
        _gating_kernel,
        out_shape=jax.ShapeDtypeStruct((_T, _E), jnp.float32),
    )(x, wg)

    moe_out = pl.pallas_call(
        _moe_kernel,
        grid=(_E, _NF),
        in_specs=[
            pl.BlockSpec((_T, _E), lambda e, f: (0, 0)),
            pl.BlockSpec((_T, _D), lambda e, f: (0, 0)),
            pl.BlockSpec((1, _D, _FB), lambda e, f: (e, 0, f)),
            pl.BlockSpec((1, _D, _FB), lambda e, f: (e, 0, _NF + f)),
            pl.BlockSpec((1, _FB, _D), lambda e, f: (e, f, 0)),
        ],
        out_specs=pl.BlockSpec((_T, _D), lambda e, f: (0, 0)),
        out_shape=jax.ShapeDtypeStruct((_T, _D), jnp.float32),
    )(comb, x, W_gu_exp, W_gu_exp, W_down_exp)

    shared_out = pl.pallas_call(
        _shared_kernel,
        grid=(_NF,),
        in_specs=[
            pl.BlockSpec((_T, _D), lambda f: (0, 0)),
            pl.BlockSpec((_D, _FB), lambda f: (0, f)),
            pl.BlockSpec((_D, _FB), lambda f: (0, _NF + f)),
            pl.BlockSpec((_FB, _D), lambda f: (f, 0)),
        ],
        out_specs=pl.BlockSpec((_T, _D), lambda f: (0, 0)),
        out_shape=jax.ShapeDtypeStruct((_T, _D), jnp.float32),
    )(x, W_gu_shared, W_gu_shared, W_down_shared)

    return moe_out + shared_out


# speedup vs baseline: 2.5469x; 1.4953x over previous
import jax
import jax.numpy as jnp
from jax.experimental import pallas as pl
from jax.experimental.pallas import tpu as pltpu

_T, _D, _E, _K, _I = 2048, 768, 64, 8, 6144
_H = _I // 2
_FB = 512
_NF = _H // _FB
_B = 128
_GMAX = _T * _K // _B + _E


def _gating_kernel(x_ref, wg_ref, eidx_ref, p_ref):
    x = x_ref[...]
    wg = wg_ref[...]
    logits = jax.lax.dot_general(
        x, wg, (((1,), (1,)), ((), ())), preferred_element_type=jnp.float32)
    m = jnp.max(logits, axis=1, keepdims=True)
    ex = jnp.exp(logits - m)
    gates = ex / jnp.sum(ex, axis=1, keepdims=True)

    g = gates
    iota = jax.lax.broadcasted_iota(jnp.int32, (_T, _E), 1)
    iota_k = jax.lax.broadcasted_iota(jnp.int32, (_T, _K), 1)
    eidx = jnp.zeros((_T, _K), jnp.int32)
    tops = jnp.zeros((_T, _K), jnp.float32)
    topsum = jnp.zeros((_T, 1), jnp.float32)
    for k in range(_K):
        mx = jnp.max(g, axis=1, keepdims=True)
        sel = g == mx
        first = jnp.min(jnp.where(sel, iota, _E), axis=1, keepdims=True)
        onehot = iota == first
        eidx = jnp.where(iota_k == k, first, eidx)
        tops = jnp.where(iota_k == k, mx, tops)
        topsum = topsum + mx
        g = jnp.where(onehot, -jnp.inf, g)
    denom = jnp.maximum(topsum, jnp.finfo(jnp.float32).eps)
    eidx_ref[...] = eidx
    p_ref[...] = tops / denom


def _moe_kernel(e_sm, a_sm, x_ref, toks_ref, tokl_ref, w_ref,
                wgate_ref, wup_ref, wdn_ref, out_ref):
    f = pl.program_id(0)
    g = pl.program_id(1)

    @pl.when(jnp.logical_and(f == 0, g == 0))
    def _():
        out_ref[...] = jnp.zeros_like(out_ref)

    @pl.when(a_sm[g] == 1)
    def _():
        x = x_ref[...]
        tok_col = toks_ref[0]
        iota_l = jax.lax.broadcasted_iota(jnp.int32, (_B, _T), 1)
        gather_m = (iota_l == tok_col).astype(jnp.float32)
        xs = jax.lax.dot(gather_m, x, preferred_element_type=jnp.float32)

        x1 = jax.lax.dot(xs, wgate_ref[0],
                         preferred_element_type=jnp.float32)
        x2 = jax.lax.dot(xs, wup_ref[0], preferred_element_type=jnp.float32)
        act = x1 * (x2 * jax.nn.sigmoid(x2))
        ys = jax.lax.dot(act, wdn_ref[0], preferred_element_type=jnp.float32)

        tok_row = tokl_ref[0]
        w_row = w_ref[0]
        iota_s = jax.lax.broadcasted_iota(jnp.int32, (_T, _B), 0)
        scatter_m = jnp.where(iota_s == tok_row, w_row, 0.0)
        out_ref[...] += jax.lax.dot(scatter_m, ys,
                                    preferred_element_type=jnp.float32)


def _shared_kernel(x_ref, wg_ref, wu_ref, wd_ref, out_ref):
    f = pl.program_id(0)

    @pl.when(f == 0)
    def _():
        out_ref[...] = jnp.zeros_like(out_ref)

    x = x_ref[...]
    x1 = jax.lax.dot(x, wg_ref[...], preferred_element_type=jnp.float32)
    x2 = jax.lax.dot(x, wu_ref[...], preferred_element_type=jnp.float32)
    act = x1 * (x2 * jax.nn.sigmoid(x2))
    out_ref[...] += jax.lax.dot(act, wd_ref[...],
                                preferred_element_type=jnp.float32)


def kernel(x, wg, W_gu_shared, W_down_shared, W_gu_exp, W_down_exp):
    eidx, p = pl.pallas_call(
        _gating_kernel,
        out_shape=(jax.ShapeDtypeStruct((_T, _K), jnp.int32),
                   jax.ShapeDtypeStruct((_T, _K), jnp.float32)),
    )(x, wg)

    ei = eidx.reshape(-1)
    wf = p.reshape(-1)
    tok = jnp.arange(_T * _K, dtype=jnp.int32) // _K
    order = jnp.argsort(ei)
    tok_sorted = tok[order]
    w_sorted = wf[order]
    counts = jnp.bincount(ei, length=_E)
    offs = jnp.cumsum(counts) - counts
    nblk = (counts + _B - 1) // _B
    cumblk = jnp.cumsum(nblk)
    total = cumblk[_E - 1]
    gidx = jnp.arange(_GMAX, dtype=jnp.int32)
    gc = jnp.minimum(gidx, total - 1)
    e_g = jnp.searchsorted(cumblk, gc, side='right').astype(jnp.int32)
    active = (gidx < total).astype(jnp.int32)
    row0 = offs[e_g] + (gc - (cumblk[e_g] - nblk[e_g])) * _B
    rows = row0[:, None] + jnp.arange(_B, dtype=jnp.int32)[None, :]
    valid = (rows < (offs[e_g] + counts[e_g])[:, None]) & (active[:, None] == 1)
    safe = jnp.where(valid, rows, 0)
    tok_blk = tok_sorted[safe].astype(jnp.int32)
    w_blk = jnp.where(valid, w_sorted[safe], 0.0)

    hw = _H // 2
    moe_out = pl.pallas_call(
        _moe_kernel,
        grid_spec=pltpu.PrefetchScalarGridSpec(
            num_scalar_prefetch=2,
            grid=(2, _GMAX),
            in_specs=[
                pl.BlockSpec((_T, _D), lambda f, g, es, as_: (0, 0)),
                pl.BlockSpec((1, _B, 1), lambda f, g, es, as_: (g, 0, 0)),
                pl.BlockSpec((1, 1, _B), lambda f, g, es, as_: (g, 0, 0)),
                pl.BlockSpec((1, 1, _B), lambda f, g, es, as_: (g, 0, 0)),
                pl.BlockSpec((1, _D, hw),
                             lambda f, g, es, as_: (es[g], 0, f)),
                pl.BlockSpec((1, _D, hw),
                             lambda f, g, es, as_: (es[g], 0, 2 + f)),
                pl.BlockSpec((1, hw, _D),
                             lambda f, g, es, as_: (es[g], f, 0)),
            ],
            out_specs=pl.BlockSpec((_T, _D), lambda f, g, es, as_: (0, 0)),
        ),
        out_shape=jax.ShapeDtypeStruct((_T, _D), jnp.float32),
        compiler_params=pltpu.CompilerParams(
            dimension_semantics=("arbitrary", "arbitrary")),
    )(e_g, active, x, tok_blk.reshape(_GMAX, _B, 1),
      tok_blk.reshape(_GMAX, 1, _B), w_blk.reshape(_GMAX, 1, _B),
      W_gu_exp, W_gu_exp, W_down_exp)

    shared_out = pl.pallas_call(
        _shared_kernel,
        grid=(_NF,),
        in_specs=[
            pl.BlockSpec((_T, _D), lambda f: (0, 0)),
            pl.BlockSpec((_D, _FB), lambda f: (0, f)),
            pl.BlockSpec((_D, _FB), lambda f: (0, _NF + f)),
            pl.BlockSpec((_FB, _D), lambda f: (f, 0)),
        ],
        out_specs=pl.BlockSpec((_T, _D), lambda f: (0, 0)),
        out_shape=jax.ShapeDtypeStruct((_T, _D), jnp.float32),
    )(x, W_gu_shared, W_gu_shared, W_down_shared)

    return moe_out + shared_out
